# baseline (speedup 1.0000x reference)
import jax
import jax.numpy as jnp
from jax import lax
from jax.experimental import pallas as pl
from jax.experimental.pallas import tpu as pltpu
from jax.experimental.pallas import tpu_sc as plsc

NC = 2
NS = 16
NW = NC * NS

C_JS = 400
C_SUB = 80
E_BLK = 1000
E_SUB = 40


def kernel(a, c, e, edge_index, t, batch, W_atom, W_charge, W_edge, W_time, W_count):
    n_nodes = a.shape[0]
    n_edges = e.shape[0]
    n_graphs = t.shape[0]
    d_model = W_atom.shape[1]
    d_edge = W_edge.shape[1]
    max_count = W_count.shape[0]

    cnt_len = ((n_graphs + 15) // 16) * 16
    c_njobs = n_nodes // C_JS
    c_rounds = (c_njobs + NW - 1) // NW
    e_per_w = n_edges // NW
    e_blocks = e_per_w // E_BLK
    search_steps = max(1, (n_nodes - 1).bit_length())

    def body(a_h, c_h, e_h, ei_h, t_h, b_h, wa_h, wc_h, we_h, wt_h, wn_h,
             h0_h, ei0_h, ei1_h, ee_h,
             batch_v, cnt_ref, t_ref,
             idx_a, idx_c, idx_n, idx_t, rows, idx_e, erows, we_shr, sem):
        ci = lax.axis_index("c")
        si = lax.axis_index("s")
        wid = si * NC + ci
        i16 = lax.iota(jnp.int32, 16)

        pltpu.sync_copy(b_h, batch_v)
        pltpu.sync_copy(t_h, t_ref.at[pl.ds(0, n_graphs)])

        def searchsorted(g):
            lo = jnp.zeros((16,), jnp.int32)
            hi = jnp.full((16,), n_nodes, jnp.int32)

            def step(_, lh):
                lo, hi = lh
                mid = (lo + hi) // 2
                probe = plsc.load_gather(
                    batch_v, [jnp.minimum(mid, jnp.int32(n_nodes - 1))])
                act = lo < hi
                pred = probe < g
                lo = jnp.where(act & pred, mid + 1, lo)
                hi = jnp.where(act & jnp.logical_not(pred), mid, hi)
                return (lo, hi)

            lo, hi = lax.fori_loop(0, search_steps, step, (lo, hi))
            return lo

        def cnt_block(k, _):
            g = 16 * k + i16
            cnt_ref[pl.ds(16 * k, 16)] = jnp.minimum(
                searchsorted(g + 1) - searchsorted(g),
                jnp.int32(max_count - 1))
            return 0

        lax.fori_loop(0, cnt_len // 16, cnt_block, 0)

        for r in range(c_rounds):
            job = wid + NW * r

            @pl.when(job < c_njobs)
            def _():
                base = job * C_JS
                pltpu.sync_copy(a_h.at[pl.ds(base, C_JS)], idx_a)
                pltpu.sync_copy(c_h.at[pl.ds(base, C_JS)], idx_c)
                for v in range(C_JS // 16):
                    bv = batch_v[pl.ds(base + 16 * v, 16)]
                    idx_n[pl.ds(16 * v, 16)] = plsc.load_gather(cnt_ref, [bv])
                    idx_t[pl.ds(16 * v, 16)] = plsc.load_gather(t_ref, [bv])
                for tbl, ib, col in ((wa_h, idx_a, 0), (wc_h, idx_c, 1),
                                     (wn_h, idx_n, 2), (wt_h, idx_t, 3)):
                    descs = []
                    for s in range(C_JS // C_SUB):
                        descs.append(pltpu.async_copy(
                            tbl.at[ib.at[pl.ds(C_SUB * s, C_SUB)]],
                            rows.at[pl.ds(C_SUB * s, C_SUB)], sem))
                    for d in descs:
                        d.wait()
                    pltpu.sync_copy(
                        rows,
                        h0_h.at[pl.ds(base, C_JS),
                                pl.ds(d_model * col, d_model)])

        @pl.when(si == 0)
        def _():
            pltpu.sync_copy(we_h, we_shr)

        plsc.subcore_barrier()
        ebase = wid * e_per_w
        for blk in range(e_blocks):
            off = ebase + E_BLK * blk
            pltpu.sync_copy(e_h.at[pl.ds(off, E_BLK)], idx_e)
            descs = []
            for s in range(E_BLK // E_SUB):
                descs.append(pltpu.async_copy(
                    we_shr.at[idx_e.at[pl.ds(E_SUB * s, E_SUB)]],
                    erows.at[pl.ds(E_SUB * s, E_SUB)], sem))
            for d in descs:
                d.wait()
            pltpu.sync_copy(erows, ee_h.at[pl.ds(off, E_BLK)])

        pltpu.sync_copy(ei_h.at[pl.ds(ebase, e_per_w)],
                        ei0_h.at[pl.ds(ebase, e_per_w)])
        pltpu.sync_copy(ei_h.at[pl.ds(n_edges + ebase, e_per_w)],
                        ei1_h.at[pl.ds(ebase, e_per_w)])

    mesh = plsc.VectorSubcoreMesh(core_axis_name="c", subcore_axis_name="s")
    out_type = (
        jax.ShapeDtypeStruct((n_nodes, 4 * d_model), jnp.float32),
        jax.ShapeDtypeStruct((n_edges,), edge_index.dtype),
        jax.ShapeDtypeStruct((n_edges,), edge_index.dtype),
        jax.ShapeDtypeStruct((n_edges, d_edge), jnp.float32),
    )
    scratch = [
        pltpu.VMEM((n_nodes,), jnp.int32),
        pltpu.VMEM((cnt_len,), jnp.int32),
        pltpu.VMEM((cnt_len,), jnp.int32),
        pltpu.VMEM((C_JS,), jnp.int32),
        pltpu.VMEM((C_JS,), jnp.int32),
        pltpu.VMEM((C_JS,), jnp.int32),
        pltpu.VMEM((C_JS,), jnp.int32),
        pltpu.VMEM((C_JS, d_model), jnp.float32),
        pltpu.VMEM((E_BLK,), jnp.int32),
        pltpu.VMEM((E_BLK, d_edge), jnp.float32),
        pltpu.VMEM_SHARED((5, 16), jnp.float32),
        pltpu.SemaphoreType.DMA,
    ]
    run = pl.kernel(
        body, out_type=out_type, mesh=mesh, scratch_types=scratch,
        compiler_params=pltpu.CompilerParams(
            needs_layout_passes=False, use_tc_tiling_on_sc=False))
    h0, ei0, ei1, eemb = run(a, c, e, edge_index.reshape(-1), t, batch,
                             W_atom, W_charge, W_edge, W_time, W_count)
    return h0, (ei0, ei1), eemb


# speedup vs baseline: 1.1872x; 1.1872x over previous
import jax
import jax.numpy as jnp
from jax import lax
from jax.experimental import pallas as pl
from jax.experimental.pallas import tpu as pltpu
from jax.experimental.pallas import tpu_sc as plsc

NC = 2
NS = 16
NW = NC * NS

C_JS = 400
C_SUB = 80
E_BLK = 1000
E_SUB = 40


def kernel(a, c, e, edge_index, t, batch, W_atom, W_charge, W_edge, W_time, W_count):
    n_nodes = a.shape[0]
    n_edges = e.shape[0]
    n_graphs = t.shape[0]
    d_model = W_atom.shape[1]
    d_edge = W_edge.shape[1]
    max_count = W_count.shape[0]

    cnt_len = ((n_graphs + 15) // 16) * 16
    c_njobs = n_nodes // C_JS
    c_rounds = (c_njobs + NW - 1) // NW
    e_per_w = n_edges // NW
    e_blocks = e_per_w // E_BLK
    search_steps = max(1, (n_nodes - 1).bit_length())

    def body(a_h, c_h, e_h, ei_h, t_h, b_h, wa_h, wc_h, we_h, wt_h, wn_h,
             h0_h, ei0_h, ei1_h, ee_h,
             batch_v, cnt_ref, t_ref,
             idx_a, idx_c, idx_n, idx_t, rows, idx_e, erows,
             we_shr, wa_shr, wc_shr, wn_shr, sem):
        ci = lax.axis_index("c")
        si = lax.axis_index("s")
        wid = si * NC + ci
        i16 = lax.iota(jnp.int32, 16)

        pltpu.sync_copy(b_h, batch_v)
        pltpu.sync_copy(t_h, t_ref.at[pl.ds(0, n_graphs)])

        def searchsorted(g):
            lo = jnp.zeros((16,), jnp.int32)
            hi = jnp.full((16,), n_nodes, jnp.int32)

            def step(_, lh):
                lo, hi = lh
                mid = (lo + hi) // 2
                probe = plsc.load_gather(
                    batch_v, [jnp.minimum(mid, jnp.int32(n_nodes - 1))])
                act = lo < hi
                pred = probe < g
                lo = jnp.where(act & pred, mid + 1, lo)
                hi = jnp.where(act & jnp.logical_not(pred), mid, hi)
                return (lo, hi)

            lo, hi = lax.fori_loop(0, search_steps, step, (lo, hi))
            return lo

        def cnt_block(k, _):
            g = 16 * k + i16
            cnt_ref[pl.ds(16 * k, 16)] = jnp.minimum(
                searchsorted(g + 1) - searchsorted(g),
                jnp.int32(max_count - 1))
            return 0

        lax.fori_loop(0, cnt_len // 16, cnt_block, 0)

        @pl.when(si == 0)
        def _():
            pltpu.sync_copy(we_h, we_shr)
            pltpu.sync_copy(wa_h, wa_shr)
            pltpu.sync_copy(wc_h, wc_shr)

        @pl.when(si == 1)
        def _():
            pltpu.sync_copy(wn_h, wn_shr)

        plsc.subcore_barrier()

        for r in range(c_rounds):
            job = wid + NW * r

            @pl.when(job < c_njobs)
            def _():
                base = job * C_JS
                pltpu.sync_copy(a_h.at[pl.ds(base, C_JS)], idx_a)
                pltpu.sync_copy(c_h.at[pl.ds(base, C_JS)], idx_c)
                for v in range(C_JS // 16):
                    bv = batch_v[pl.ds(base + 16 * v, 16)]
                    idx_n[pl.ds(16 * v, 16)] = plsc.load_gather(cnt_ref, [bv])
                    idx_t[pl.ds(16 * v, 16)] = plsc.load_gather(t_ref, [bv])
                for tbl, ib, col in ((wa_shr, idx_a, 0), (wc_shr, idx_c, 1),
                                     (wn_shr, idx_n, 2), (wt_h, idx_t, 3)):
                    descs = []
                    for s in range(C_JS // C_SUB):
                        descs.append(pltpu.async_copy(
                            tbl.at[ib.at[pl.ds(C_SUB * s, C_SUB)]],
                            rows.at[pl.ds(C_SUB * s, C_SUB)], sem))
                    for d in descs:
                        d.wait()
                    pltpu.sync_copy(
                        rows,
                        h0_h.at[pl.ds(base, C_JS),
                                pl.ds(d_model * col, d_model)])

        ebase = wid * e_per_w
        for blk in range(e_blocks):
            off = ebase + E_BLK * blk
            pltpu.sync_copy(e_h.at[pl.ds(off, E_BLK)], idx_e)
            descs = []
            for s in range(E_BLK // E_SUB):
                descs.append(pltpu.async_copy(
                    we_shr.at[idx_e.at[pl.ds(E_SUB * s, E_SUB)]],
                    erows.at[pl.ds(E_SUB * s, E_SUB)], sem))
            for d in descs:
                d.wait()
            pltpu.sync_copy(erows, ee_h.at[pl.ds(off, E_BLK)])

        pltpu.sync_copy(ei_h.at[pl.ds(ebase, e_per_w)],
                        ei0_h.at[pl.ds(ebase, e_per_w)])
        pltpu.sync_copy(ei_h.at[pl.ds(n_edges + ebase, e_per_w)],
                        ei1_h.at[pl.ds(ebase, e_per_w)])

    mesh = plsc.VectorSubcoreMesh(core_axis_name="c", subcore_axis_name="s")
    out_type = (
        jax.ShapeDtypeStruct((n_nodes, 4 * d_model), jnp.float32),
        jax.ShapeDtypeStruct((n_edges,), edge_index.dtype),
        jax.ShapeDtypeStruct((n_edges,), edge_index.dtype),
        jax.ShapeDtypeStruct((n_edges, d_edge), jnp.float32),
    )
    scratch = [
        pltpu.VMEM((n_nodes,), jnp.int32),
        pltpu.VMEM((cnt_len,), jnp.int32),
        pltpu.VMEM((cnt_len,), jnp.int32),
        pltpu.VMEM((C_JS,), jnp.int32),
        pltpu.VMEM((C_JS,), jnp.int32),
        pltpu.VMEM((C_JS,), jnp.int32),
        pltpu.VMEM((C_JS,), jnp.int32),
        pltpu.VMEM((C_JS, d_model), jnp.float32),
        pltpu.VMEM((E_BLK,), jnp.int32),
        pltpu.VMEM((E_BLK, d_edge), jnp.float32),
        pltpu.VMEM_SHARED((5, 16), jnp.float32),
        pltpu.VMEM_SHARED(W_atom.shape, jnp.float32),
        pltpu.VMEM_SHARED(W_charge.shape, jnp.float32),
        pltpu.VMEM_SHARED(W_count.shape, jnp.float32),
        pltpu.SemaphoreType.DMA,
    ]
    run = pl.kernel(
        body, out_type=out_type, mesh=mesh, scratch_types=scratch,
        compiler_params=pltpu.CompilerParams(
            needs_layout_passes=False, use_tc_tiling_on_sc=False))
    h0, ei0, ei1, eemb = run(a, c, e, edge_index.reshape(-1), t, batch,
                             W_atom, W_charge, W_edge, W_time, W_count)
    return h0, (ei0, ei1), eemb


# speedup vs baseline: 1.1930x; 1.0049x over previous
import jax
import jax.numpy as jnp
from jax import lax
from jax.experimental import pallas as pl
from jax.experimental.pallas import tpu as pltpu
from jax.experimental.pallas import tpu_sc as plsc

NC = 2
NS = 16
NW = NC * NS

C_JS = 400
C_SUB = 80
E_BLK = 1000
E_SUB = 40


def kernel(a, c, e, edge_index, t, batch, W_atom, W_charge, W_edge, W_time, W_count):
    n_nodes = a.shape[0]
    n_edges = e.shape[0]
    n_graphs = t.shape[0]
    d_model = W_atom.shape[1]
    d_edge = W_edge.shape[1]
    max_count = W_count.shape[0]

    cnt_len = ((n_graphs + 15) // 16) * 16
    c_njobs = n_nodes // C_JS
    c_rounds = (c_njobs + NW - 1) // NW
    e_per_w = n_edges // NW
    e_blocks = e_per_w // E_BLK
    search_steps = max(1, (n_nodes - 1).bit_length())

    def body(a_h, c_h, e_h, ei_h, t_h, b_h, wa_h, wc_h, we_h, wt_h, wn_h,
             h0_h, ei0_h, ei1_h, ee_h,
             batch_v, cnt_ref, t_ref,
             idx_a, idx_c, idx_n, idx_t, rows, idx_e, erows,
             we_shr, wa_shr, wc_shr, wn_shr, sem):
        ci = lax.axis_index("c")
        si = lax.axis_index("s")
        wid = si * NC + ci
        i16 = lax.iota(jnp.int32, 16)

        pltpu.sync_copy(b_h, batch_v)
        pltpu.sync_copy(t_h, t_ref.at[pl.ds(0, n_graphs)])

        def searchsorted(g):
            lo = jnp.zeros((16,), jnp.int32)
            hi = jnp.full((16,), n_nodes, jnp.int32)

            def step(_, lh):
                lo, hi = lh
                mid = (lo + hi) // 2
                probe = plsc.load_gather(
                    batch_v, [jnp.minimum(mid, jnp.int32(n_nodes - 1))])
                act = lo < hi
                pred = probe < g
                lo = jnp.where(act & pred, mid + 1, lo)
                hi = jnp.where(act & jnp.logical_not(pred), mid, hi)
                return (lo, hi)

            lo, hi = lax.fori_loop(0, search_steps, step, (lo, hi))
            return lo

        def cnt_block(k, _):
            g = 16 * k + i16
            cnt_ref[pl.ds(16 * k, 16)] = jnp.minimum(
                searchsorted(g + 1) - searchsorted(g),
                jnp.int32(max_count - 1))
            return 0

        lax.fori_loop(0, cnt_len // 16, cnt_block, 0)

        @pl.when(si == 0)
        def _():
            pltpu.sync_copy(we_h, we_shr)
            pltpu.sync_copy(wa_h, wa_shr)
            pltpu.sync_copy(wc_h, wc_shr)

        @pl.when(si == 1)
        def _():
            pltpu.sync_copy(wn_h, wn_shr)

        plsc.subcore_barrier()

        for r in range(c_rounds):
            job = wid + NW * r

            @pl.when(job < c_njobs)
            def _():
                base = job * C_JS
                pltpu.sync_copy(a_h.at[pl.ds(base, C_JS)], idx_a)
                pltpu.sync_copy(c_h.at[pl.ds(base, C_JS)], idx_c)
                for v in range(C_JS // 16):
                    bv = batch_v[pl.ds(base + 16 * v, 16)]
                    idx_n[pl.ds(16 * v, 16)] = plsc.load_gather(cnt_ref, [bv])
                    idx_t[pl.ds(16 * v, 16)] = plsc.load_gather(t_ref, [bv])
                for tbl, ib, col in ((wa_shr, idx_a, 0), (wc_shr, idx_c, 1),
                                     (wn_shr, idx_n, 2), (wt_h, idx_t, 3)):
                    descs = []
                    for s in range(C_JS // C_SUB):
                        descs.append(pltpu.async_copy(
                            tbl.at[ib.at[pl.ds(C_SUB * s, C_SUB)]],
                            rows.at[pl.ds(C_SUB * s, C_SUB)], sem))
                    for d in descs:
                        d.wait()
                    pltpu.sync_copy(
                        rows,
                        h0_h.at[pl.ds(base, C_JS),
                                pl.ds(d_model * col, d_model)])

        ebase = wid * e_per_w
        for blk in range(e_blocks):
            off = ebase + E_BLK * blk
            pltpu.sync_copy(e_h.at[pl.ds(off, E_BLK)], idx_e)
            descs = []
            for s in range(E_BLK // E_SUB):
                descs.append(pltpu.async_copy(
                    we_shr.at[idx_e.at[pl.ds(E_SUB * s, E_SUB)]],
                    erows.at[pl.ds(E_SUB * s, E_SUB)], sem))
            for d in descs:
                d.wait()
            pltpu.sync_copy(erows, ee_h.at[pl.ds(off, E_BLK)])

        pltpu.sync_copy(ei_h.at[pl.ds(ebase, e_per_w)],
                        ei0_h.at[pl.ds(ebase, e_per_w)])
        pltpu.sync_copy(ei_h.at[pl.ds(n_edges + ebase, e_per_w)],
                        ei1_h.at[pl.ds(ebase, e_per_w)])

    mesh = plsc.VectorSubcoreMesh(core_axis_name="c", subcore_axis_name="s")
    out_type = (
        jax.ShapeDtypeStruct((n_nodes, 4 * d_model), jnp.float32),
        jax.ShapeDtypeStruct((n_edges,), edge_index.dtype),
        jax.ShapeDtypeStruct((n_edges,), edge_index.dtype),
        jax.ShapeDtypeStruct((n_edges, d_edge), jnp.float32),
    )
    scratch = [
        pltpu.VMEM((n_nodes,), jnp.int32),
        pltpu.VMEM((cnt_len,), jnp.int32),
        pltpu.VMEM((cnt_len,), jnp.int32),
        pltpu.VMEM((C_JS,), jnp.int32),
        pltpu.VMEM((C_JS,), jnp.int32),
        pltpu.VMEM((C_JS,), jnp.int32),
        pltpu.VMEM((C_JS,), jnp.int32),
        pltpu.VMEM((C_JS, d_model), jnp.float32),
        pltpu.VMEM((E_BLK,), jnp.int32),
        pltpu.VMEM((E_BLK, d_edge), jnp.float32),
        pltpu.VMEM_SHARED(W_edge.shape, jnp.float32),
        pltpu.VMEM_SHARED(W_atom.shape, jnp.float32),
        pltpu.VMEM_SHARED(W_charge.shape, jnp.float32),
        pltpu.VMEM_SHARED(W_count.shape, jnp.float32),
        pltpu.SemaphoreType.DMA,
    ]
    run = pl.kernel(
        body, out_type=out_type, mesh=mesh, scratch_types=scratch,
        compiler_params=pltpu.CompilerParams(
            needs_layout_passes=False, use_tc_tiling_on_sc=False))
    h0, ei0, ei1, eemb = run(a, c, e, edge_index.reshape(-1), t, batch,
                             W_atom, W_charge, W_edge, W_time, W_count)
    return h0, (ei0, ei1), eemb
